# baseline (speedup 1.0000x reference)
import functools

import jax
import jax.numpy as jnp
from jax import lax
from jax.experimental import pallas as pl
from jax.experimental.pallas import tpu as pltpu
from jax.experimental.pallas import tpu_sc as plsc

N = 10000
D_IN = 128
D_HID = 128
D_OUT = 64
E = 320000

NC = 2
NS = 16
NW = NC * NS
L = 128
CH = 80
NP = 5
PCH = CH // NP
EPT = CH * L
EPAD = EPT * NW
SR = 640
NACC = SR * NS

_mesh = plsc.VectorSubcoreMesh(core_axis_name="c", subcore_axis_name="s")


@functools.partial(
    pl.kernel,
    out_type=jax.ShapeDtypeStruct((NC * NACC,), jnp.float32),
    mesh=_mesh,
    scratch_types=[
        pltpu.VMEM((CH, L), jnp.int32),
        pltpu.VMEM((SR,), jnp.float32),
        pltpu.VMEM_SHARED((NACC,), jnp.float32),
    ],
)
def _deg_sc(dst_hbm, out_hbm, idx_v, buf_v, acc):
    c = lax.axis_index("c")
    s = lax.axis_index("s")
    w = c * NS + s

    def fill0(i, carry):
        buf_v[pl.ds(i * 16, 16)] = jnp.zeros((16,), jnp.float32)
        return carry

    lax.fori_loop(0, SR // 16, fill0, 0)
    pltpu.sync_copy(buf_v, acc.at[pl.ds(s * SR, SR)])

    def fill1(i, carry):
        buf_v[pl.ds(i * 16, 16)] = jnp.ones((16,), jnp.float32)
        return carry

    lax.fori_loop(0, L // 16, fill1, 0)
    pltpu.sync_copy(dst_hbm.at[w], idx_v)
    plsc.subcore_barrier()

    def body(j, carry):
        pltpu.sync_copy(buf_v.at[pl.ds(0, L)], acc.at[idx_v.at[j]], add=True)
        return carry

    lax.fori_loop(0, CH, body, 0)
    plsc.subcore_barrier()
    pltpu.sync_copy(acc.at[pl.ds(s * SR, SR)], buf_v)
    pltpu.sync_copy(buf_v, out_hbm.at[pl.ds(c * NACC + s * SR, SR)])


def _make_edge_agg(D):
    @functools.partial(
        pl.kernel,
        out_type=jax.ShapeDtypeStruct((NC, NACC, D), jnp.float32),
        mesh=_mesh,
        scratch_types=[
            pltpu.VMEM((CH, L), jnp.int32),
            pltpu.VMEM((CH, L), jnp.int32),
            pltpu.VMEM((L, D), jnp.float32),
            pltpu.VMEM_SHARED((NACC, D), jnp.float32),
            pltpu.SemaphoreType.DMA,
        ],
    )
    def _agg(h_hbm, src_hbm, dst_hbm, out_hbm, sidx, didx, rows, acc, sem):
        c = lax.axis_index("c")
        s = lax.axis_index("s")
        w = c * NS + s

        def z(i, carry):
            r = i // (D // 16)
            col = (i % (D // 16)) * 16
            rows[r, pl.ds(col, 16)] = jnp.zeros((16,), jnp.float32)
            return carry

        lax.fori_loop(0, L * (D // 16), z, 0)
        base = s * SR
        for k in range(SR // L):
            pltpu.sync_copy(rows, acc.at[pl.ds(base + k * L, L)])
        pltpu.sync_copy(src_hbm.at[w], sidx)
        pltpu.sync_copy(dst_hbm.at[w], didx)
        plsc.subcore_barrier()

        def body(j, carry):
            pltpu.async_copy(h_hbm.at[sidx.at[j]], rows, sem).wait()
            pltpu.sync_copy(rows, acc.at[didx.at[j]], add=True)
            return carry

        lax.fori_loop(0, CH, body, 0)
        plsc.subcore_barrier()
        for k in range(SR // L):
            pltpu.sync_copy(acc.at[pl.ds(base + k * L, L)],
                            out_hbm.at[c, pl.ds(base + k * L, L)])

    return _agg


_agg128 = _make_edge_agg(D_HID)


_BR = 2000


def _mm1_body(d0_ref, d1_ref, x_ref, w_ref, h_ref, dinv_ref):
    deg = d0_ref[...] + d1_ref[...] + 1.0
    dinv = lax.rsqrt(jnp.maximum(deg, 1.0))
    dinv_ref[...] = dinv
    h = jnp.dot(x_ref[...], w_ref[...], preferred_element_type=jnp.float32)
    h_ref[...] = h * dinv


_mm1 = pl.pallas_call(
    _mm1_body,
    grid=(N // _BR,),
    in_specs=[
        pl.BlockSpec((_BR, 1), lambda i: (i, 0)),
        pl.BlockSpec((_BR, 1), lambda i: (i, 0)),
        pl.BlockSpec((_BR, D_IN), lambda i: (i, 0)),
        pl.BlockSpec((D_IN, D_HID), lambda i: (0, 0)),
    ],
    out_specs=[
        pl.BlockSpec((_BR, D_HID), lambda i: (i, 0)),
        pl.BlockSpec((_BR, 1), lambda i: (i, 0)),
    ],
    out_shape=[
        jax.ShapeDtypeStruct((N, D_HID), jnp.float32),
        jax.ShapeDtypeStruct((N, 1), jnp.float32),
    ],
)


def _mm2_body(p0_ref, p1_ref, h1_ref, dinv_ref, b1_ref, w2_ref, h2_ref):
    dinv = dinv_ref[...]
    srow = (p0_ref[...] + p1_ref[...] + h1_ref[...]) * dinv + b1_ref[...]
    z = jnp.maximum(srow, 0.0)
    h2 = jnp.dot(z, w2_ref[...], preferred_element_type=jnp.float32)
    h2_ref[...] = h2 * dinv


_mm2 = pl.pallas_call(
    _mm2_body,
    grid=(N // _BR,),
    in_specs=[
        pl.BlockSpec((_BR, D_HID), lambda i: (i, 0)),
        pl.BlockSpec((_BR, D_HID), lambda i: (i, 0)),
        pl.BlockSpec((_BR, D_HID), lambda i: (i, 0)),
        pl.BlockSpec((_BR, 1), lambda i: (i, 0)),
        pl.BlockSpec((1, D_HID), lambda i: (0, 0)),
        pl.BlockSpec((D_HID, D_HID), lambda i: (0, 0)),
    ],
    out_specs=pl.BlockSpec((_BR, D_HID), lambda i: (i, 0)),
    out_shape=jax.ShapeDtypeStruct((N, D_HID), jnp.float32),
)


def _fin_body(q0_ref, q1_ref, h2_ref, dinv_ref, b2_ref, o_ref):
    z = (q0_ref[...] + q1_ref[...] + h2_ref[...]) * dinv_ref[...] + b2_ref[...]
    m = jnp.max(z, axis=1, keepdims=True)
    lse = jnp.log(jnp.sum(jnp.exp(z - m), axis=1, keepdims=True)) + m
    o_ref[...] = z - lse


_fin = pl.pallas_call(
    _fin_body,
    grid=(N // _BR,),
    in_specs=[
        pl.BlockSpec((_BR, D_OUT), lambda i: (i, 0)),
        pl.BlockSpec((_BR, D_OUT), lambda i: (i, 0)),
        pl.BlockSpec((_BR, D_OUT), lambda i: (i, 0)),
        pl.BlockSpec((_BR, 1), lambda i: (i, 0)),
        pl.BlockSpec((1, D_OUT), lambda i: (0, 0)),
    ],
    out_specs=pl.BlockSpec((_BR, D_OUT), lambda i: (i, 0)),
    out_shape=jax.ShapeDtypeStruct((N, D_OUT), jnp.float32),
)


def kernel(x, edge_index, W1, b1, W2, b2):
    src = edge_index[0]
    dst = edge_index[1]
    npad = EPAD - E
    src3 = jnp.concatenate(
        [src, jnp.zeros((npad,), jnp.int32)]).reshape(NW, CH, L)
    dst3 = jnp.concatenate(
        [dst, jnp.full((npad,), N, jnp.int32)]).reshape(NW, CH, L)

    degp = _deg_sc(dst3).reshape(NC, NACC)
    d0 = degp[0, :N, None]
    d1 = degp[1, :N, None]
    h1p, dinv = _mm1(d0, d1, x, W1)
    p = _agg128(h1p, src3, dst3)
    W2p = jnp.pad(W2, ((0, 0), (0, D_HID - D_OUT)))
    h2p = _mm2(p[0, :N], p[1, :N], h1p, dinv, b1.reshape(1, D_HID), W2p)
    q = _agg128(h2p, src3, dst3)
    return _fin(q[0, :N, :D_OUT], q[1, :N, :D_OUT], h2p[:, :D_OUT], dinv,
                b2.reshape(1, D_OUT))


# speedup vs baseline: 2.2938x; 2.2938x over previous
import functools

import jax
import jax.numpy as jnp
from jax import lax
from jax.experimental import pallas as pl
from jax.experimental.pallas import tpu as pltpu
from jax.experimental.pallas import tpu_sc as plsc

N = 10000
D_IN = 128
D_HID = 128
D_OUT = 64
E = 320000

NC = 2
NS = 16
NW = NC * NS
L = 128
CH = 79
EPT = CH * L
EPAD = EPT * NW
SR = 640
NACC = SR * NS

_mesh = plsc.VectorSubcoreMesh(core_axis_name="c", subcore_axis_name="s")


@functools.partial(
    pl.kernel,
    out_type=jax.ShapeDtypeStruct((NC * NACC,), jnp.float32),
    mesh=_mesh,
    scratch_types=[
        pltpu.VMEM((CH, L), jnp.int32),
        pltpu.VMEM((SR,), jnp.float32),
        pltpu.VMEM_SHARED((NACC,), jnp.float32),
    ],
)
def _deg_sc(dst_hbm, out_hbm, idx_v, buf_v, acc):
    c = lax.axis_index("c")
    s = lax.axis_index("s")
    w = c * NS + s

    def fill0(i, carry):
        buf_v[pl.ds(i * 16, 16)] = jnp.zeros((16,), jnp.float32)
        return carry

    lax.fori_loop(0, SR // 16, fill0, 0)
    pltpu.sync_copy(buf_v, acc.at[pl.ds(s * SR, SR)])

    def fill1(i, carry):
        buf_v[pl.ds(i * 16, 16)] = jnp.ones((16,), jnp.float32)
        return carry

    lax.fori_loop(0, L // 16, fill1, 0)
    pltpu.sync_copy(dst_hbm.at[w], idx_v)
    plsc.subcore_barrier()

    def body(j, carry):
        pltpu.sync_copy(buf_v.at[pl.ds(0, L)], acc.at[idx_v.at[j]], add=True)
        return carry

    lax.fori_loop(0, CH, body, 0)
    plsc.subcore_barrier()
    pltpu.sync_copy(acc.at[pl.ds(s * SR, SR)], buf_v)
    pltpu.sync_copy(buf_v, out_hbm.at[pl.ds(c * NACC + s * SR, SR)])


def _make_edge_agg(D):
    @functools.partial(
        pl.kernel,
        out_type=jax.ShapeDtypeStruct((NC, NACC, D), jnp.float32),
        mesh=_mesh,
        scratch_types=[
            pltpu.VMEM((CH, L), jnp.int32),
            pltpu.VMEM((CH, L), jnp.int32),
            pltpu.VMEM((L, D), jnp.float32),
            pltpu.VMEM_SHARED((NACC, D), jnp.float32),
            pltpu.SemaphoreType.DMA,
        ],
    )
    def _agg(h_hbm, src_hbm, dst_hbm, out_hbm, sidx, didx, rows, acc, sem):
        c = lax.axis_index("c")
        s = lax.axis_index("s")
        w = c * NS + s

        def z(i, carry):
            r = i // (D // 16)
            col = (i % (D // 16)) * 16
            rows[r, pl.ds(col, 16)] = jnp.zeros((16,), jnp.float32)
            return carry

        lax.fori_loop(0, L * (D // 16), z, 0)
        base = s * SR
        for k in range(SR // L):
            pltpu.sync_copy(rows, acc.at[pl.ds(base + k * L, L)])
        pltpu.sync_copy(src_hbm.at[w], sidx)
        pltpu.sync_copy(dst_hbm.at[w], didx)
        plsc.subcore_barrier()

        def body(j, carry):
            pltpu.async_copy(h_hbm.at[sidx.at[j]], rows, sem).wait()
            pltpu.sync_copy(rows, acc.at[didx.at[j]], add=True)
            return carry

        lax.fori_loop(0, CH, body, 0)
        plsc.subcore_barrier()
        for k in range(SR // L):
            pltpu.sync_copy(acc.at[pl.ds(base + k * L, L)],
                            out_hbm.at[c, pl.ds(base + k * L, L)])

    return _agg


_agg128 = _make_edge_agg(D_HID)


_BR = 2000


def _mm1_body(d0_ref, d1_ref, x_ref, w_ref, h_ref, dinv_ref):
    deg = d0_ref[...] + d1_ref[...] + 1.0
    dinv = lax.rsqrt(jnp.maximum(deg, 1.0))
    dinv_ref[...] = dinv
    h = jnp.dot(x_ref[...], w_ref[...], preferred_element_type=jnp.float32)
    h_ref[...] = h * dinv


_mm1 = pl.pallas_call(
    _mm1_body,
    grid=(N // _BR,),
    in_specs=[
        pl.BlockSpec((_BR, 1), lambda i: (i, 0)),
        pl.BlockSpec((_BR, 1), lambda i: (i, 0)),
        pl.BlockSpec((_BR, D_IN), lambda i: (i, 0)),
        pl.BlockSpec((D_IN, D_HID), lambda i: (0, 0)),
    ],
    out_specs=[
        pl.BlockSpec((_BR, D_HID), lambda i: (i, 0)),
        pl.BlockSpec((_BR, 1), lambda i: (i, 0)),
    ],
    out_shape=[
        jax.ShapeDtypeStruct((N, D_HID), jnp.float32),
        jax.ShapeDtypeStruct((N, 1), jnp.float32),
    ],
)


def _mm2_body(p0_ref, p1_ref, h1_ref, dinv_ref, b1_ref, w2_ref, h2_ref):
    dinv = dinv_ref[...]
    srow = (p0_ref[...] + p1_ref[...] + h1_ref[...]) * dinv + b1_ref[...]
    z = jnp.maximum(srow, 0.0)
    h2 = jnp.dot(z, w2_ref[...], preferred_element_type=jnp.float32)
    h2_ref[...] = h2 * dinv


_mm2 = pl.pallas_call(
    _mm2_body,
    grid=(N // _BR,),
    in_specs=[
        pl.BlockSpec((_BR, D_HID), lambda i: (i, 0)),
        pl.BlockSpec((_BR, D_HID), lambda i: (i, 0)),
        pl.BlockSpec((_BR, D_HID), lambda i: (i, 0)),
        pl.BlockSpec((_BR, 1), lambda i: (i, 0)),
        pl.BlockSpec((1, D_HID), lambda i: (0, 0)),
        pl.BlockSpec((D_HID, D_HID), lambda i: (0, 0)),
    ],
    out_specs=pl.BlockSpec((_BR, D_HID), lambda i: (i, 0)),
    out_shape=jax.ShapeDtypeStruct((N, D_HID), jnp.float32),
)


def _fin_body(q0_ref, q1_ref, h2_ref, dinv_ref, b2_ref, o_ref):
    z = (q0_ref[...] + q1_ref[...] + h2_ref[...]) * dinv_ref[...] + b2_ref[...]
    m = jnp.max(z, axis=1, keepdims=True)
    lse = jnp.log(jnp.sum(jnp.exp(z - m), axis=1, keepdims=True)) + m
    o_ref[...] = z - lse


_fin = pl.pallas_call(
    _fin_body,
    grid=(N // _BR,),
    in_specs=[
        pl.BlockSpec((_BR, D_OUT), lambda i: (i, 0)),
        pl.BlockSpec((_BR, D_OUT), lambda i: (i, 0)),
        pl.BlockSpec((_BR, D_OUT), lambda i: (i, 0)),
        pl.BlockSpec((_BR, 1), lambda i: (i, 0)),
        pl.BlockSpec((1, D_OUT), lambda i: (0, 0)),
    ],
    out_specs=pl.BlockSpec((_BR, D_OUT), lambda i: (i, 0)),
    out_shape=jax.ShapeDtypeStruct((N, D_OUT), jnp.float32),
)


def kernel(x, edge_index, W1, b1, W2, b2):
    src = edge_index[0]
    dst = edge_index[1]
    npad = EPAD - E
    ar = jnp.arange(npad, dtype=jnp.int32)
    src3 = jnp.concatenate([src, ar % N]).reshape(NW, CH, L)
    dst3 = jnp.concatenate([dst, N + (ar % (NACC - N))]).reshape(NW, CH, L)

    degp = _deg_sc(dst3).reshape(NC, NACC)
    d0 = degp[0, :N, None]
    d1 = degp[1, :N, None]
    h1p, dinv = _mm1(d0, d1, x, W1)
    p = _agg128(h1p, src3, dst3)
    W2p = jnp.pad(W2, ((0, 0), (0, D_HID - D_OUT)))
    h2p = _mm2(p[0, :N], p[1, :N], h1p, dinv, b1.reshape(1, D_HID), W2p)
    q = _agg128(h2p, src3, dst3)
    return _fin(q[0, :N, :D_OUT], q[1, :N, :D_OUT], h2p[:, :D_OUT], dinv,
                b2.reshape(1, D_OUT))


# speedup vs baseline: 3.0456x; 1.3277x over previous
import functools

import jax
import jax.numpy as jnp
from jax import lax
from jax.experimental import pallas as pl
from jax.experimental.pallas import tpu as pltpu
from jax.experimental.pallas import tpu_sc as plsc

N = 10000
D_IN = 128
D_HID = 128
D_OUT = 64
E = 320000

NC = 2
NS = 16
NW = NC * NS
L = 128
CH = 80
NP = 5
PCH = CH // NP
EPT = CH * L
EPAD = EPT * NW
SR = 640
NACC = SR * NS

_mesh = plsc.VectorSubcoreMesh(core_axis_name="c", subcore_axis_name="s")


@functools.partial(
    pl.kernel,
    out_type=jax.ShapeDtypeStruct((NC * NACC,), jnp.float32),
    mesh=_mesh,
    scratch_types=[
        pltpu.VMEM((CH, L), jnp.int32),
        pltpu.VMEM((SR,), jnp.float32),
        pltpu.VMEM_SHARED((NACC,), jnp.float32),
    ],
)
def _deg_sc(dst_hbm, out_hbm, idx_v, buf_v, acc):
    c = lax.axis_index("c")
    s = lax.axis_index("s")
    w = c * NS + s

    def fill0(i, carry):
        buf_v[pl.ds(i * 16, 16)] = jnp.zeros((16,), jnp.float32)
        return carry

    lax.fori_loop(0, SR // 16, fill0, 0)
    pltpu.sync_copy(buf_v, acc.at[pl.ds(s * SR, SR)])

    def fill1(i, carry):
        buf_v[pl.ds(i * 16, 16)] = jnp.ones((16,), jnp.float32)
        return carry

    lax.fori_loop(0, L // 16, fill1, 0)
    pltpu.sync_copy(dst_hbm.at[w], idx_v)
    plsc.subcore_barrier()

    def body(j, carry):
        pltpu.sync_copy(buf_v.at[pl.ds(0, L)], acc.at[idx_v.at[j]], add=True)
        return carry

    lax.fori_loop(0, CH, body, 0)
    plsc.subcore_barrier()
    pltpu.sync_copy(acc.at[pl.ds(s * SR, SR)], buf_v)
    pltpu.sync_copy(buf_v, out_hbm.at[pl.ds(c * NACC + s * SR, SR)])


def _make_edge_agg(D):
    @functools.partial(
        pl.kernel,
        out_type=jax.ShapeDtypeStruct((NC, NACC, D), jnp.float32),
        mesh=_mesh,
        scratch_types=[
            pltpu.VMEM((PCH, L), jnp.int32),
            pltpu.VMEM((PCH, L), jnp.int32),
            pltpu.VMEM((L, D), jnp.float32),
            pltpu.VMEM((L, D), jnp.float32),
            pltpu.VMEM_SHARED((NACC, D), jnp.float32),
            pltpu.SemaphoreType.DMA,
            pltpu.SemaphoreType.DMA,
        ],
    )
    def _agg(h_hbm, src_hbm, dst_hbm, out_hbm, sidx, didx,
             rows_a, rows_b, acc, ga, gb):
        c = lax.axis_index("c")
        s = lax.axis_index("s")
        w = c * NS + s

        def z(i, carry):
            r = i // (D // 16)
            col = (i % (D // 16)) * 16
            rows_a[r, pl.ds(col, 16)] = jnp.zeros((16,), jnp.float32)
            return carry

        lax.fori_loop(0, L * (D // 16), z, 0)
        base = s * SR
        for k in range(SR // L):
            pltpu.sync_copy(rows_a, acc.at[pl.ds(base + k * L, L)])
        plsc.subcore_barrier()

        def gather(si, buf, sem):
            pltpu.async_copy(h_hbm.at[si], buf, sem)

        def gather_wait(si, buf, sem):
            pltpu.make_async_copy(h_hbm.at[si], buf, sem).wait()

        def scat(di, buf):
            pltpu.sync_copy(buf, acc.at[di], add=True)

        for half in range(NP):
            pltpu.sync_copy(src_hbm.at[w, pl.ds(half * PCH, PCH)], sidx)
            pltpu.sync_copy(dst_hbm.at[w, pl.ds(half * PCH, PCH)], didx)
            gather(sidx.at[0], rows_a, ga)

            def body(k, carry):
                e = 2 * k
                gather(sidx.at[e + 1], rows_b, gb)
                gather_wait(sidx.at[e], rows_a, ga)
                scat(didx.at[e], rows_a)
                gather(sidx.at[e + 2], rows_a, ga)
                gather_wait(sidx.at[e + 1], rows_b, gb)
                scat(didx.at[e + 1], rows_b)
                return carry

            lax.fori_loop(0, PCH // 2 - 1, body, 0)
            e = PCH - 2
            gather(sidx.at[e + 1], rows_b, gb)
            gather_wait(sidx.at[e], rows_a, ga)
            scat(didx.at[e], rows_a)
            gather_wait(sidx.at[e + 1], rows_b, gb)
            scat(didx.at[e + 1], rows_b)
        plsc.subcore_barrier()
        for k in range(SR // L):
            pltpu.sync_copy(acc.at[pl.ds(base + k * L, L)],
                            out_hbm.at[c, pl.ds(base + k * L, L)])

    return _agg


_agg128 = _make_edge_agg(D_HID)


_BR = 2000


def _mm1_body(d0_ref, d1_ref, x_ref, w_ref, h_ref, dinv_ref):
    deg = d0_ref[...] + d1_ref[...] + 1.0
    dinv = lax.rsqrt(jnp.maximum(deg, 1.0))
    dinv_ref[...] = dinv
    h = jnp.dot(x_ref[...], w_ref[...], preferred_element_type=jnp.float32)
    h_ref[...] = h * dinv


_mm1 = pl.pallas_call(
    _mm1_body,
    grid=(N // _BR,),
    in_specs=[
        pl.BlockSpec((_BR, 1), lambda i: (i, 0)),
        pl.BlockSpec((_BR, 1), lambda i: (i, 0)),
        pl.BlockSpec((_BR, D_IN), lambda i: (i, 0)),
        pl.BlockSpec((D_IN, D_HID), lambda i: (0, 0)),
    ],
    out_specs=[
        pl.BlockSpec((_BR, D_HID), lambda i: (i, 0)),
        pl.BlockSpec((_BR, 1), lambda i: (i, 0)),
    ],
    out_shape=[
        jax.ShapeDtypeStruct((N, D_HID), jnp.float32),
        jax.ShapeDtypeStruct((N, 1), jnp.float32),
    ],
)


def _mm2_body(p0_ref, p1_ref, h1_ref, dinv_ref, b1_ref, w2_ref, h2_ref):
    dinv = dinv_ref[...]
    srow = (p0_ref[...] + p1_ref[...] + h1_ref[...]) * dinv + b1_ref[...]
    z = jnp.maximum(srow, 0.0)
    h2 = jnp.dot(z, w2_ref[...], preferred_element_type=jnp.float32)
    h2_ref[...] = h2 * dinv


_mm2 = pl.pallas_call(
    _mm2_body,
    grid=(N // _BR,),
    in_specs=[
        pl.BlockSpec((_BR, D_HID), lambda i: (i, 0)),
        pl.BlockSpec((_BR, D_HID), lambda i: (i, 0)),
        pl.BlockSpec((_BR, D_HID), lambda i: (i, 0)),
        pl.BlockSpec((_BR, 1), lambda i: (i, 0)),
        pl.BlockSpec((1, D_HID), lambda i: (0, 0)),
        pl.BlockSpec((D_HID, D_HID), lambda i: (0, 0)),
    ],
    out_specs=pl.BlockSpec((_BR, D_HID), lambda i: (i, 0)),
    out_shape=jax.ShapeDtypeStruct((N, D_HID), jnp.float32),
)


def _fin_body(q0_ref, q1_ref, h2_ref, dinv_ref, b2_ref, o_ref):
    z = (q0_ref[...] + q1_ref[...] + h2_ref[...]) * dinv_ref[...] + b2_ref[...]
    m = jnp.max(z, axis=1, keepdims=True)
    lse = jnp.log(jnp.sum(jnp.exp(z - m), axis=1, keepdims=True)) + m
    o_ref[...] = z - lse


_fin = pl.pallas_call(
    _fin_body,
    grid=(N // _BR,),
    in_specs=[
        pl.BlockSpec((_BR, D_OUT), lambda i: (i, 0)),
        pl.BlockSpec((_BR, D_OUT), lambda i: (i, 0)),
        pl.BlockSpec((_BR, D_OUT), lambda i: (i, 0)),
        pl.BlockSpec((_BR, 1), lambda i: (i, 0)),
        pl.BlockSpec((1, D_OUT), lambda i: (0, 0)),
    ],
    out_specs=pl.BlockSpec((_BR, D_OUT), lambda i: (i, 0)),
    out_shape=jax.ShapeDtypeStruct((N, D_OUT), jnp.float32),
)


def kernel(x, edge_index, W1, b1, W2, b2):
    src = edge_index[0]
    dst = edge_index[1]
    npad = EPAD - E
    ar = jnp.arange(npad, dtype=jnp.int32)
    src3 = jnp.concatenate([src, ar % N]).reshape(NW, CH, L)
    dst3 = jnp.concatenate([dst, N + (ar % (NACC - N))]).reshape(NW, CH, L)

    degp = _deg_sc(dst3).reshape(NC, NACC)
    d0 = degp[0, :N, None]
    d1 = degp[1, :N, None]
    h1p, dinv = _mm1(d0, d1, x, W1)
    p = _agg128(h1p, src3, dst3)
    W2p = jnp.pad(W2, ((0, 0), (0, D_HID - D_OUT)))
    h2p = _mm2(p[0, :N], p[1, :N], h1p, dinv, b1.reshape(1, D_HID), W2p)
    q = _agg128(h2p, src3, dst3)
    return _fin(q[0, :N, :D_OUT], q[1, :N, :D_OUT], h2p[:, :D_OUT], dinv,
                b2.reshape(1, D_OUT))


# speedup vs baseline: 3.1735x; 1.0420x over previous
import functools

import jax
import jax.numpy as jnp
from jax import lax
from jax.experimental import pallas as pl
from jax.experimental.pallas import tpu as pltpu
from jax.experimental.pallas import tpu_sc as plsc

N = 10000
D_IN = 128
D_HID = 128
D_OUT = 64
E = 320000

NC = 2
NS = 16
NW = NC * NS
L = 128
CH = 80
NP = 5
PCH = CH // NP
EPT = CH * L
EPAD = EPT * NW
SR = 640
NACC = SR * NS

_mesh = plsc.VectorSubcoreMesh(core_axis_name="c", subcore_axis_name="s")


@functools.partial(
    pl.kernel,
    out_type=jax.ShapeDtypeStruct((NC * NACC,), jnp.float32),
    mesh=_mesh,
    scratch_types=[
        pltpu.VMEM((CH, L), jnp.int32),
        pltpu.VMEM((SR,), jnp.float32),
        pltpu.VMEM_SHARED((NACC,), jnp.float32),
    ],
)
def _deg_sc(dst_hbm, out_hbm, idx_v, buf_v, acc):
    c = lax.axis_index("c")
    s = lax.axis_index("s")
    w = c * NS + s

    def fill0(i, carry):
        buf_v[pl.ds(i * 16, 16)] = jnp.zeros((16,), jnp.float32)
        return carry

    lax.fori_loop(0, SR // 16, fill0, 0)
    pltpu.sync_copy(buf_v, acc.at[pl.ds(s * SR, SR)])

    def fill1(i, carry):
        buf_v[pl.ds(i * 16, 16)] = jnp.ones((16,), jnp.float32)
        return carry

    lax.fori_loop(0, L // 16, fill1, 0)
    pltpu.sync_copy(dst_hbm.at[w], idx_v)
    plsc.subcore_barrier()

    def body(j, carry):
        pltpu.sync_copy(buf_v.at[pl.ds(0, L)], acc.at[idx_v.at[j]], add=True)
        return carry

    lax.fori_loop(0, CH, body, 0)
    plsc.subcore_barrier()
    pltpu.sync_copy(acc.at[pl.ds(s * SR, SR)], buf_v)
    pltpu.sync_copy(buf_v, out_hbm.at[pl.ds(c * NACC + s * SR, SR)])


def _make_edge_agg(D):
    @functools.partial(
        pl.kernel,
        out_type=jax.ShapeDtypeStruct((NC, NACC, D), jnp.float32),
        mesh=_mesh,
        scratch_types=[
            pltpu.VMEM((PCH, L), jnp.int32),
            pltpu.VMEM((PCH, L), jnp.int32),
            pltpu.VMEM((L, D), jnp.float32),
            pltpu.VMEM((L, D), jnp.float32),
            pltpu.VMEM_SHARED((NACC, D), jnp.float32),
            pltpu.SemaphoreType.DMA,
            pltpu.SemaphoreType.DMA,
        ],
    )
    def _agg(h_hbm, src_hbm, dst_hbm, out_hbm, sidx, didx,
             rows_a, rows_b, acc, ga, gb):
        c = lax.axis_index("c")
        s = lax.axis_index("s")
        w = c * NS + s

        def z(i, carry):
            r = i // (D // 16)
            col = (i % (D // 16)) * 16
            rows_a[r, pl.ds(col, 16)] = jnp.zeros((16,), jnp.float32)
            return carry

        lax.fori_loop(0, L * (D // 16), z, 0)
        base = s * SR
        for k in range(SR // L):
            pltpu.sync_copy(rows_a, acc.at[pl.ds(base + k * L, L)])
        plsc.subcore_barrier()

        def gather(si, buf, sem):
            pltpu.async_copy(h_hbm.at[si], buf, sem)

        def gather_wait(si, buf, sem):
            pltpu.make_async_copy(h_hbm.at[si], buf, sem).wait()

        def scat(di, buf):
            pltpu.sync_copy(buf, acc.at[di], add=True)

        for half in range(NP):
            pltpu.sync_copy(src_hbm.at[w, pl.ds(half * PCH, PCH)], sidx)
            pltpu.sync_copy(dst_hbm.at[w, pl.ds(half * PCH, PCH)], didx)
            gather(sidx.at[0], rows_a, ga)

            def body(k, carry):
                e = 2 * k
                gather(sidx.at[e + 1], rows_b, gb)
                gather_wait(sidx.at[e], rows_a, ga)
                scat(didx.at[e], rows_a)
                gather(sidx.at[e + 2], rows_a, ga)
                gather_wait(sidx.at[e + 1], rows_b, gb)
                scat(didx.at[e + 1], rows_b)
                return carry

            lax.fori_loop(0, PCH // 2 - 1, body, 0)
            e = PCH - 2
            gather(sidx.at[e + 1], rows_b, gb)
            gather_wait(sidx.at[e], rows_a, ga)
            scat(didx.at[e], rows_a)
            gather_wait(sidx.at[e + 1], rows_b, gb)
            scat(didx.at[e + 1], rows_b)
        plsc.subcore_barrier()
        for k in range(SR // L):
            pltpu.sync_copy(acc.at[pl.ds(base + k * L, L)],
                            out_hbm.at[c, pl.ds(base + k * L, L)])

    return _agg


_agg128 = _make_edge_agg(D_HID)


_BR = 2000


def _mm1_body(d0_ref, d1_ref, x_ref, w_ref, h_ref, dinv_ref):
    deg = d0_ref[...] + d1_ref[...] + 1.0
    dinv = lax.rsqrt(jnp.maximum(deg, 1.0))
    dinv_ref[...] = dinv
    h = jnp.dot(x_ref[...], w_ref[...], preferred_element_type=jnp.float32)
    h_ref[...] = h * dinv


_mm1 = pl.pallas_call(
    _mm1_body,
    grid=(N // _BR,),
    in_specs=[
        pl.BlockSpec((_BR, 1), lambda i: (i, 0)),
        pl.BlockSpec((_BR, 1), lambda i: (i, 0)),
        pl.BlockSpec((_BR, D_IN), lambda i: (i, 0)),
        pl.BlockSpec((D_IN, D_HID), lambda i: (0, 0)),
    ],
    out_specs=[
        pl.BlockSpec((_BR, D_HID), lambda i: (i, 0)),
        pl.BlockSpec((_BR, 1), lambda i: (i, 0)),
    ],
    out_shape=[
        jax.ShapeDtypeStruct((N, D_HID), jnp.float32),
        jax.ShapeDtypeStruct((N, 1), jnp.float32),
    ],
)


def _mm2_body(p_ref, h1_ref, dinv_ref, b1_ref, w2_ref, h2_ref):
    dinv = dinv_ref[...]
    srow = (p_ref[0] + p_ref[1] + h1_ref[...]) * dinv + b1_ref[...]
    z = jnp.maximum(srow, 0.0)
    h2 = jnp.dot(z, w2_ref[...], preferred_element_type=jnp.float32)
    h2_ref[...] = h2 * dinv


_mm2 = pl.pallas_call(
    _mm2_body,
    grid=(N // _BR,),
    in_specs=[
        pl.BlockSpec((NC, _BR, D_HID), lambda i: (0, i, 0)),
        pl.BlockSpec((_BR, D_HID), lambda i: (i, 0)),
        pl.BlockSpec((_BR, 1), lambda i: (i, 0)),
        pl.BlockSpec((1, D_HID), lambda i: (0, 0)),
        pl.BlockSpec((D_HID, D_HID), lambda i: (0, 0)),
    ],
    out_specs=pl.BlockSpec((_BR, D_HID), lambda i: (i, 0)),
    out_shape=jax.ShapeDtypeStruct((N, D_HID), jnp.float32),
)


def _fin_body(q_ref, h2_ref, dinv_ref, b2_ref, o_ref):
    z = ((q_ref[0, :, :D_OUT] + q_ref[1, :, :D_OUT] + h2_ref[..., :D_OUT])
         * dinv_ref[...] + b2_ref[...])
    m = jnp.max(z, axis=1, keepdims=True)
    lse = jnp.log(jnp.sum(jnp.exp(z - m), axis=1, keepdims=True)) + m
    o_ref[...] = z - lse


_fin = pl.pallas_call(
    _fin_body,
    grid=(N // _BR,),
    in_specs=[
        pl.BlockSpec((NC, _BR, D_HID), lambda i: (0, i, 0)),
        pl.BlockSpec((_BR, D_HID), lambda i: (i, 0)),
        pl.BlockSpec((_BR, 1), lambda i: (i, 0)),
        pl.BlockSpec((1, D_OUT), lambda i: (0, 0)),
    ],
    out_specs=pl.BlockSpec((_BR, D_OUT), lambda i: (i, 0)),
    out_shape=jax.ShapeDtypeStruct((N, D_OUT), jnp.float32),
)


def kernel(x, edge_index, W1, b1, W2, b2):
    src = edge_index[0]
    dst = edge_index[1]
    npad = EPAD - E
    ar = jnp.arange(npad, dtype=jnp.int32)
    src3 = jnp.concatenate([src, ar % N]).reshape(NW, CH, L)
    dst3 = jnp.concatenate([dst, N + (ar % (NACC - N))]).reshape(NW, CH, L)

    degp = _deg_sc(dst3).reshape(NC, NACC)
    d0 = degp[0, :N, None]
    d1 = degp[1, :N, None]
    h1p, dinv = _mm1(d0, d1, x, W1)
    p = _agg128(h1p, src3, dst3)
    W2p = jnp.pad(W2, ((0, 0), (0, D_HID - D_OUT)))
    h2p = _mm2(p, h1p, dinv, b1.reshape(1, D_HID), W2p)
    q = _agg128(h2p, src3, dst3)
    return _fin(q, h2p, dinv, b2.reshape(1, D_OUT))


# speedup vs baseline: 3.3183x; 1.0456x over previous
import functools

import jax
import jax.numpy as jnp
from jax import lax
from jax.experimental import pallas as pl
from jax.experimental.pallas import tpu as pltpu
from jax.experimental.pallas import tpu_sc as plsc

N = 10000
D_IN = 128
D_HID = 128
D_OUT = 64
E = 320000

NC = 2
NS = 16
NW = NC * NS
L = 128
CH = 80
NP = 5
PCH = CH // NP
EPT = CH * L
EPAD = EPT * NW
SR = 640
NACC = SR * NS

_mesh = plsc.VectorSubcoreMesh(core_axis_name="c", subcore_axis_name="s")


@functools.partial(
    pl.kernel,
    out_type=jax.ShapeDtypeStruct((NC * NACC,), jnp.float32),
    mesh=_mesh,
    scratch_types=[
        pltpu.VMEM((CH, L), jnp.int32),
        pltpu.VMEM((SR,), jnp.float32),
        pltpu.VMEM_SHARED((NACC,), jnp.float32),
    ],
)
def _deg_sc(dst_hbm, out_hbm, idx_v, buf_v, acc):
    c = lax.axis_index("c")
    s = lax.axis_index("s")
    w = c * NS + s

    def fill0(i, carry):
        buf_v[pl.ds(i * 16, 16)] = jnp.zeros((16,), jnp.float32)
        return carry

    lax.fori_loop(0, SR // 16, fill0, 0)
    pltpu.sync_copy(buf_v, acc.at[pl.ds(s * SR, SR)])

    def fill1(i, carry):
        buf_v[pl.ds(i * 16, 16)] = jnp.ones((16,), jnp.float32)
        return carry

    lax.fori_loop(0, L // 16, fill1, 0)
    pltpu.sync_copy(dst_hbm.at[w], idx_v)
    plsc.subcore_barrier()

    def body(j, carry):
        pltpu.sync_copy(buf_v.at[pl.ds(0, L)], acc.at[idx_v.at[j]], add=True)
        return carry

    lax.fori_loop(0, CH, body, 0)
    plsc.subcore_barrier()
    pltpu.sync_copy(acc.at[pl.ds(s * SR, SR)], buf_v)
    pltpu.sync_copy(buf_v, out_hbm.at[pl.ds(c * NACC + s * SR, SR)])


def _make_edge_agg(D):
    @functools.partial(
        pl.kernel,
        out_type=jax.ShapeDtypeStruct((NC, NACC, D), jnp.float32),
        mesh=_mesh,
        scratch_types=[
            pltpu.VMEM((PCH, L), jnp.int32),
            pltpu.VMEM((PCH, L), jnp.int32),
            pltpu.VMEM((PCH, L), jnp.int32),
            pltpu.VMEM((PCH, L), jnp.int32),
            pltpu.VMEM((L, D), jnp.float32),
            pltpu.VMEM((L, D), jnp.float32),
            pltpu.VMEM_SHARED((NACC, D), jnp.float32),
            pltpu.SemaphoreType.DMA,
            pltpu.SemaphoreType.DMA,
        ],
    )
    def _agg(h_hbm, src_hbm, dst_hbm, out_hbm, sidx0, didx0, sidx1, didx1,
             rows_a, rows_b, acc, ga, gb):
        c = lax.axis_index("c")
        s = lax.axis_index("s")
        w = c * NS + s

        def z(i, carry):
            r = i // (D // 16)
            col = (i % (D // 16)) * 16
            rows_a[r, pl.ds(col, 16)] = jnp.zeros((16,), jnp.float32)
            return carry

        lax.fori_loop(0, L * (D // 16), z, 0)
        base = s * SR
        for k in range(SR // L):
            pltpu.sync_copy(rows_a, acc.at[pl.ds(base + k * L, L)])
        plsc.subcore_barrier()

        def gather(si, buf, sem):
            pltpu.async_copy(h_hbm.at[si], buf, sem)

        def gather_wait(si, buf, sem):
            pltpu.make_async_copy(h_hbm.at[si], buf, sem).wait()

        def scat(di, buf):
            pltpu.sync_copy(buf, acc.at[di], add=True)

        banks = ((sidx0, didx0), (sidx1, didx1))
        pltpu.sync_copy(src_hbm.at[w, pl.ds(0, PCH)], sidx0)
        pltpu.sync_copy(dst_hbm.at[w, pl.ds(0, PCH)], didx0)
        gather(sidx0.at[0], rows_a, ga)
        for p in range(NP):
            sidx, didx = banks[p % 2]
            nsidx, ndidx = banks[(p + 1) % 2]
            if p + 1 < NP:
                pltpu.sync_copy(src_hbm.at[w, pl.ds((p + 1) * PCH, PCH)], nsidx)
                pltpu.sync_copy(dst_hbm.at[w, pl.ds((p + 1) * PCH, PCH)], ndidx)

            def body(k, carry):
                e = 2 * k
                gather(sidx.at[e + 1], rows_b, gb)
                gather_wait(sidx.at[e], rows_a, ga)
                scat(didx.at[e], rows_a)
                gather(sidx.at[e + 2], rows_a, ga)
                gather_wait(sidx.at[e + 1], rows_b, gb)
                scat(didx.at[e + 1], rows_b)
                return carry

            lax.fori_loop(0, PCH // 2 - 1, body, 0)
            e = PCH - 2
            gather(sidx.at[e + 1], rows_b, gb)
            gather_wait(sidx.at[e], rows_a, ga)
            scat(didx.at[e], rows_a)
            if p + 1 < NP:
                gather(nsidx.at[0], rows_a, ga)
            gather_wait(sidx.at[e + 1], rows_b, gb)
            scat(didx.at[e + 1], rows_b)
        plsc.subcore_barrier()
        for k in range(SR // L):
            pltpu.sync_copy(acc.at[pl.ds(base + k * L, L)],
                            out_hbm.at[c, pl.ds(base + k * L, L)])

    return _agg


_agg128 = _make_edge_agg(D_HID)


_BR = 2000


def _mm1_body(d0_ref, d1_ref, x_ref, w_ref, h_ref, dinv_ref):
    deg = d0_ref[...] + d1_ref[...] + 1.0
    dinv = lax.rsqrt(jnp.maximum(deg, 1.0))
    dinv_ref[...] = dinv
    h = jnp.dot(x_ref[...], w_ref[...], preferred_element_type=jnp.float32)
    h_ref[...] = h * dinv


_mm1 = pl.pallas_call(
    _mm1_body,
    grid=(N // _BR,),
    in_specs=[
        pl.BlockSpec((_BR, 1), lambda i: (i, 0)),
        pl.BlockSpec((_BR, 1), lambda i: (i, 0)),
        pl.BlockSpec((_BR, D_IN), lambda i: (i, 0)),
        pl.BlockSpec((D_IN, D_HID), lambda i: (0, 0)),
    ],
    out_specs=[
        pl.BlockSpec((_BR, D_HID), lambda i: (i, 0)),
        pl.BlockSpec((_BR, 1), lambda i: (i, 0)),
    ],
    out_shape=[
        jax.ShapeDtypeStruct((N, D_HID), jnp.float32),
        jax.ShapeDtypeStruct((N, 1), jnp.float32),
    ],
)


def _mm2_body(p_ref, h1_ref, dinv_ref, b1_ref, w2_ref, h2_ref):
    dinv = dinv_ref[...]
    srow = (p_ref[0] + p_ref[1] + h1_ref[...]) * dinv + b1_ref[...]
    z = jnp.maximum(srow, 0.0)
    h2 = jnp.dot(z, w2_ref[...], preferred_element_type=jnp.float32)
    h2_ref[...] = h2 * dinv


_mm2 = pl.pallas_call(
    _mm2_body,
    grid=(N // _BR,),
    in_specs=[
        pl.BlockSpec((NC, _BR, D_HID), lambda i: (0, i, 0)),
        pl.BlockSpec((_BR, D_HID), lambda i: (i, 0)),
        pl.BlockSpec((_BR, 1), lambda i: (i, 0)),
        pl.BlockSpec((1, D_HID), lambda i: (0, 0)),
        pl.BlockSpec((D_HID, D_HID), lambda i: (0, 0)),
    ],
    out_specs=pl.BlockSpec((_BR, D_HID), lambda i: (i, 0)),
    out_shape=jax.ShapeDtypeStruct((N, D_HID), jnp.float32),
)


def _fin_body(q_ref, h2_ref, dinv_ref, b2_ref, o_ref):
    z = ((q_ref[0, :, :D_OUT] + q_ref[1, :, :D_OUT] + h2_ref[..., :D_OUT])
         * dinv_ref[...] + b2_ref[...])
    m = jnp.max(z, axis=1, keepdims=True)
    lse = jnp.log(jnp.sum(jnp.exp(z - m), axis=1, keepdims=True)) + m
    o_ref[...] = z - lse


_fin = pl.pallas_call(
    _fin_body,
    grid=(N // _BR,),
    in_specs=[
        pl.BlockSpec((NC, _BR, D_HID), lambda i: (0, i, 0)),
        pl.BlockSpec((_BR, D_HID), lambda i: (i, 0)),
        pl.BlockSpec((_BR, 1), lambda i: (i, 0)),
        pl.BlockSpec((1, D_OUT), lambda i: (0, 0)),
    ],
    out_specs=pl.BlockSpec((_BR, D_OUT), lambda i: (i, 0)),
    out_shape=jax.ShapeDtypeStruct((N, D_OUT), jnp.float32),
)


def kernel(x, edge_index, W1, b1, W2, b2):
    src = edge_index[0]
    dst = edge_index[1]
    npad = EPAD - E
    ar = jnp.arange(npad, dtype=jnp.int32)
    src3 = jnp.concatenate([src, ar % N]).reshape(NW, CH, L)
    dst3 = jnp.concatenate([dst, N + (ar % (NACC - N))]).reshape(NW, CH, L)

    degp = _deg_sc(dst3).reshape(NC, NACC)
    d0 = degp[0, :N, None]
    d1 = degp[1, :N, None]
    h1p, dinv = _mm1(d0, d1, x, W1)
    p = _agg128(h1p, src3, dst3)
    W2p = jnp.pad(W2, ((0, 0), (0, D_HID - D_OUT)))
    h2p = _mm2(p, h1p, dinv, b1.reshape(1, D_HID), W2p)
    q = _agg128(h2p, src3, dst3)
    return _fin(q, h2p, dinv, b2.reshape(1, D_OUT))
